# baseline (speedup 1.0000x reference)
import jax
import jax.numpy as jnp
from jax import lax
from jax.experimental import pallas as pl
from jax.experimental.pallas import tpu as pltpu
from jax.experimental.pallas import tpu_sc as plsc
from jax._src.pallas import mpmd as _mpmd

_B_BLK = 8
_W_BLK = 2048

_NC = 2
_ROWS_PER_TILE = 8


def _copy_body(cache_ref, out_ref):
    out_ref[...] = cache_ref[...]


def _tc_copy(cache):
    b, w, h = cache.shape
    spec = pl.BlockSpec((_B_BLK, _W_BLK, h), lambda i, j: (i, j, 0))
    return pl.pallas_call(
        _copy_body,
        grid=(b // _B_BLK, w // _W_BLK),
        out_shape=jax.ShapeDtypeStruct(cache.shape, cache.dtype),
        in_specs=[spec],
        out_specs=spec,
    )(cache)


def _scatter_body(n_tiles, rows_hbm, idx_hbm, buf_hbm, out_hbm, rows_v, idx_v, sem):
    del buf_hbm
    wid = lax.axis_index("s") * _NC + lax.axis_index("c")

    @pl.when(wid < n_tiles)
    def _():
        base = wid * _ROWS_PER_TILE
        pltpu.sync_copy(rows_hbm.at[pl.ds(base, _ROWS_PER_TILE)], rows_v)
        pltpu.sync_copy(idx_hbm.at[pl.ds(base, _ROWS_PER_TILE)], idx_v)
        pltpu.async_copy(rows_v, out_hbm.at[idx_v], sem).wait()


def _sc_scatter(rows, idx, buf_flat):
    n_rows, head = rows.shape
    n_tiles = n_rows // _ROWS_PER_TILE
    mesh = plsc.VectorSubcoreMesh(core_axis_name="c", subcore_axis_name="s")
    body = lambda *refs: _scatter_body(n_tiles, *refs)
    call = _mpmd.mpmd._mpmd_map if hasattr(_mpmd, "mpmd") else _mpmd._mpmd_map
    fn = call(
        [(mesh, body)],
        out_types=jax.ShapeDtypeStruct(buf_flat.shape, buf_flat.dtype),
        input_output_aliases={2: 0},
        scratch_types=[
            pltpu.VMEM((_ROWS_PER_TILE, head), jnp.float32),
            pltpu.VMEM((_ROWS_PER_TILE,), jnp.int32),
            pltpu.SemaphoreType.DMA,
        ],
    )
    return fn(rows, idx, buf_flat)


def kernel(kv, start_pos, kv_cache):
    bsz, _, head = kv.shape
    win = kv_cache.shape[1]
    pos = jnp.asarray(start_pos, jnp.int32) % win
    idx = jnp.arange(bsz, dtype=jnp.int32) * win + pos
    rows = jnp.squeeze(kv, axis=1)
    buf = _tc_copy(kv_cache[:bsz])
    out_flat = _sc_scatter(rows, idx, jnp.reshape(buf, (bsz * win, head)))
    return jnp.reshape(out_flat, (bsz, win, head))


# speedup vs baseline: 1.1059x; 1.1059x over previous
import jax
import jax.numpy as jnp
from jax.experimental import pallas as pl
from jax.experimental.pallas import tpu as pltpu

_B_BLK = 2


def _body(pos_ref, kv_ref, cache_ref, out_ref):
    out_ref[...] = cache_ref[...]
    out_ref[:, pl.ds(pos_ref[0], 1), :] = kv_ref[...]


def kernel(kv, start_pos, kv_cache):
    bsz, _, head = kv.shape
    win = kv_cache.shape[1]
    pos = jnp.reshape(jnp.asarray(start_pos, jnp.int32) % win, (1,))
    cache = kv_cache[:bsz]
    out = pl.pallas_call(
        _body,
        grid=(bsz // _B_BLK,),
        out_shape=jax.ShapeDtypeStruct(cache.shape, cache.dtype),
        in_specs=[
            pl.BlockSpec(memory_space=pltpu.SMEM),
            pl.BlockSpec((_B_BLK, 1, head), lambda i: (i, 0, 0)),
            pl.BlockSpec((_B_BLK, win, head), lambda i: (i, 0, 0)),
        ],
        out_specs=pl.BlockSpec((_B_BLK, win, head), lambda i: (i, 0, 0)),
    )(pos, kv, cache)
    return out
